# baseline (speedup 1.0000x reference)
import functools

import jax
import jax.numpy as jnp
from jax import lax
from jax.experimental import pallas as pl
from jax.experimental.pallas import tpu as pltpu, tpu_sc as plsc

DIM = 64
N_ROWS = 1000000
N_TILES = N_ROWS // 8
B_I, B_J = 16384, 26
B_TOTAL = B_I * B_J

_info = plsc.get_sparse_core_info()
_NC, _NS = _info.num_cores, _info.num_subcores
_NW = _NC * _NS


_K1_BASE = N_TILES // _NW
_K1_CHUNK = 31
_K1_NCH = _K1_BASE // _K1_CHUNK


def _widen_body(w3, q3, vin, vout, sem_i, sem_o):
    wid = lax.axis_index("s") * _NC + lax.axis_index("c")
    t0 = wid * _K1_BASE

    def start_in(c, b, nt=_K1_CHUNK):
        pltpu.async_copy(
            w3.at[pl.ds(t0 + c * _K1_CHUNK, nt)],
            vin.at[b, pl.ds(0, nt)],
            sem_i.at[b],
        )

    def wait_in(b, nt=_K1_CHUNK):
        pltpu.make_async_copy(
            w3.at[pl.ds(0, nt)], vin.at[b, pl.ds(0, nt)], sem_i.at[b]
        ).wait()

    def start_out(c, b, nt=_K1_CHUNK):
        pltpu.async_copy(
            vout.at[b, pl.ds(0, nt)],
            q3.at[pl.ds(t0 + c * _K1_CHUNK, nt)],
            sem_o.at[b],
        )

    def wait_out(b, nt=_K1_CHUNK):
        pltpu.make_async_copy(
            vout.at[b, pl.ds(0, nt)], q3.at[pl.ds(0, nt)], sem_o.at[b]
        ).wait()

    def widen_chunk(b, nt=_K1_CHUNK):
        def tile(t, carry):
            for s in range(8):
                for w in range(4):
                    vout[b, t, s, pl.ds(w * 16, 16)] = vin[
                        b, t, s, pl.ds(w * 16, 16)
                    ]
            return carry

        lax.fori_loop(0, nt, tile, 0)

    start_in(0, 0)

    def chunk(c, carry):
        b = lax.rem(c, 2)
        wait_in(b)

        @pl.when(c + 1 < _K1_NCH)
        def _():
            start_in(c + 1, 1 - b)

        @pl.when(c >= 2)
        def _():
            wait_out(b)

        widen_chunk(b)
        start_out(c, b)
        return carry

    lax.fori_loop(0, _K1_NCH, chunk, 0)
    wait_out(lax.rem(_K1_NCH - 2, 2))
    wait_out(lax.rem(_K1_NCH - 1, 2))

    @pl.when(wid == 0)
    def _():
        rem = N_TILES - _NW * _K1_BASE
        pltpu.async_copy(
            w3.at[pl.ds(_NW * _K1_BASE, rem)],
            vin.at[0, pl.ds(0, rem)],
            sem_i.at[0],
        ).wait()
        widen_chunk(0, rem)
        pltpu.async_copy(
            vout.at[0, pl.ds(0, rem)],
            q3.at[pl.ds(_NW * _K1_BASE, rem)],
            sem_o.at[0],
        ).wait()


_BLOCKS = B_J * (B_I // 128)
_BPW = _BLOCKS // _NW


def _gather_body(
    idxj, q2, o5, vidx0, vidx1, vrows0, vrows1, ot0, ot1, sem_i, sem_g, sem_o
):
    wid = lax.axis_index("s") * _NC + lax.axis_index("c")
    vidx = (vidx0, vidx1)
    vrows = (vrows0, vrows1)
    otile = (ot0, ot1)
    lanes = lax.iota(jnp.int32, 16)

    def fire_idx(m, b):
        bid = wid * _BPW + m
        pltpu.async_copy(idxj.at[pl.ds(bid * 128, 128)], vidx[b], sem_i.at[b])

    def wait_idx(b):
        pltpu.make_async_copy(
            idxj.at[pl.ds(0, 128)], vidx[b], sem_i.at[b]
        ).wait()

    def start_gather(b):
        pltpu.async_copy(q2.at[vidx[b]], vrows[b], sem_g.at[b])

    def wait_gather(b):
        pltpu.make_async_copy(
            q2.at[pl.ds(0, 128)], vrows[b], sem_g.at[b]
        ).wait()

    def wait_out(b):
        pltpu.make_async_copy(
            otile[b], o5.at[0, pl.ds(0, 8), pl.ds(0, 1)], sem_o.at[b]
        ).wait()

    fire_idx(0, 0)
    wait_idx(0)
    start_gather(0)
    fire_idx(1, 1)

    def run_block(m, b):
        wait_gather(b)
        bn = 1 - b

        @pl.when(m + 1 < _BPW)
        def _():
            wait_idx(bn)
            start_gather(bn)

        @pl.when(m + 2 < _BPW)
        def _():
            fire_idx(m + 2, b)

        @pl.when(m >= 2)
        def _():
            wait_out(b)

        rows = vrows[b]
        ot = otile[b]

        def col(k, carry2):
            a = k // 8
            bb = lax.rem(k, 8)
            for g in range(8):
                vals = plsc.load_gather(rows, [g * 16 + lanes, lanes * 0 + k])
                ot[a, 0, bb, pl.ds(g * 16, 16)] = vals
            return carry2

        lax.fori_loop(0, 64, col, 0, unroll=2)

        bid = wid * _BPW + m
        j = bid // 128
        ti = lax.rem(bid, 128)
        pltpu.async_copy(
            ot, o5.at[j, pl.ds(0, 8), pl.ds(ti, 1)], sem_o.at[b]
        )

    def pair(p, carry):
        run_block(2 * p, 0)
        run_block(2 * p + 1, 1)
        return carry

    lax.fori_loop(0, _BPW // 2, pair, 0)
    wait_out(0)
    wait_out(1)


@jax.jit
def kernel(token_ids, weight):
    w3 = weight.reshape(N_TILES, 8, DIM)
    widen = functools.partial(
        pl.kernel,
        out_type=jax.ShapeDtypeStruct((N_TILES, 8, 128), jnp.float32),
        mesh=plsc.VectorSubcoreMesh(core_axis_name="c", subcore_axis_name="s"),
        scratch_types=[
            pltpu.VMEM((2, _K1_CHUNK, 8, DIM), jnp.float32),
            pltpu.VMEM((2, _K1_CHUNK, 8, 128), jnp.float32),
            pltpu.SemaphoreType.DMA((2,)),
            pltpu.SemaphoreType.DMA((2,)),
        ],
        compiler_params=pltpu.CompilerParams(use_tc_tiling_on_sc=True),
    )(_widen_body)
    q3 = widen(w3)

    idxj = jnp.transpose(token_ids).reshape(B_TOTAL)
    gather = functools.partial(
        pl.kernel,
        out_type=jax.ShapeDtypeStruct((B_J, 8, 128, 8, 128), jnp.float32),
        mesh=plsc.VectorSubcoreMesh(core_axis_name="c", subcore_axis_name="s"),
        scratch_types=[
            pltpu.VMEM((128,), jnp.int32),
            pltpu.VMEM((128,), jnp.int32),
            pltpu.VMEM((128, 128), jnp.float32),
            pltpu.VMEM((128, 128), jnp.float32),
            pltpu.VMEM((8, 1, 8, 128), jnp.float32),
            pltpu.VMEM((8, 1, 8, 128), jnp.float32),
            pltpu.SemaphoreType.DMA((2,)),
            pltpu.SemaphoreType.DMA((2,)),
            pltpu.SemaphoreType.DMA((2,)),
        ],
        compiler_params=pltpu.CompilerParams(
            use_tc_tiling_on_sc=True, needs_layout_passes=False
        ),
    )(_gather_body)
    o5 = gather(idxj, q3.reshape(N_ROWS, 128))
    return o5.transpose(2, 4, 0, 1, 3).reshape(B_I, B_J, DIM)


# speedup vs baseline: 1.2422x; 1.2422x over previous
import functools

import jax
import jax.numpy as jnp
from jax import lax
from jax.experimental import pallas as pl
from jax.experimental.pallas import tpu as pltpu, tpu_sc as plsc

DIM = 64
N_ROWS = 1000000
N_TILES = N_ROWS // 8
B_I, B_J = 16384, 26
B_TOTAL = B_I * B_J

_info = plsc.get_sparse_core_info()
_NC, _NS = _info.num_cores, _info.num_subcores
_NW = _NC * _NS


_K1_BASE = N_TILES // _NW
_K1_CHUNK = 31
_K1_NCH = _K1_BASE // _K1_CHUNK


def _widen_body(w3, q3, vin0, vin1, vout0, vout1, sem_i, sem_o):
    wid = lax.axis_index("s") * _NC + lax.axis_index("c")
    t0 = wid * _K1_BASE
    vin = (vin0, vin1)
    vout = (vout0, vout1)

    def start_in(c, b):
        pltpu.async_copy(
            w3.at[pl.ds(t0 + c * _K1_CHUNK, _K1_CHUNK)], vin[b], sem_i.at[b]
        )

    def wait_in(b):
        pltpu.make_async_copy(
            w3.at[pl.ds(0, _K1_CHUNK)], vin[b], sem_i.at[b]
        ).wait()

    def start_out(c, b):
        pltpu.async_copy(
            vout[b], q3.at[pl.ds(t0 + c * _K1_CHUNK, _K1_CHUNK)], sem_o.at[b]
        )

    def wait_out(b):
        pltpu.make_async_copy(
            vout[b], q3.at[pl.ds(0, _K1_CHUNK)], sem_o.at[b]
        ).wait()

    def widen_chunk(b, nt=_K1_CHUNK):
        vi, vo = vin[b], vout[b]

        def tile(t, carry):
            for s in range(8):
                for w in range(4):
                    vo[t, s, pl.ds(w * 16, 16)] = vi[t, s, pl.ds(w * 16, 16)]
            return carry

        lax.fori_loop(0, nt, tile, 0)

    start_in(0, 0)

    def pair(p, carry):
        for b in (0, 1):
            c = 2 * p + b
            wait_in(b)

            @pl.when(c + 1 < _K1_NCH)
            def _():
                start_in(c + 1, 1 - b)

            @pl.when(c >= 2)
            def _():
                wait_out(b)

            widen_chunk(b)
            start_out(c, b)
        return carry

    lax.fori_loop(0, _K1_NCH // 2, pair, 0)
    wait_out(0)
    wait_out(1)

    @pl.when(wid == 0)
    def _():
        rem = N_TILES - _NW * _K1_BASE
        pltpu.async_copy(
            w3.at[pl.ds(_NW * _K1_BASE, rem)],
            vin[0].at[pl.ds(0, rem)],
            sem_i.at[0],
        ).wait()
        widen_chunk(0, rem)
        pltpu.async_copy(
            vout[0].at[pl.ds(0, rem)],
            q3.at[pl.ds(_NW * _K1_BASE, rem)],
            sem_o.at[0],
        ).wait()


_BLOCKS = B_J * (B_I // 128)
_BPW = _BLOCKS // _NW


def _gather_body(
    idxj, q2, o5, vidx0, vidx1, vrows0, vrows1, ot0, ot1, sem_i, sem_g, sem_o
):
    wid = lax.axis_index("s") * _NC + lax.axis_index("c")
    vidx = (vidx0, vidx1)
    vrows = (vrows0, vrows1)
    otile = (ot0, ot1)
    lanes = lax.iota(jnp.int32, 16)
    zeros = lanes * 0
    kvecs = [w * 16 + lanes for w in range(4)]
    a_idx = [kv // 8 for kv in kvecs]
    bb_idx = [lax.rem(kv, 8) for kv in kvecs]

    def fire_idx(m, b):
        bid = wid * _BPW + m
        pltpu.async_copy(idxj.at[pl.ds(bid * 128, 128)], vidx[b], sem_i.at[b])

    def wait_idx(b):
        pltpu.make_async_copy(
            idxj.at[pl.ds(0, 128)], vidx[b], sem_i.at[b]
        ).wait()

    def start_gather(b):
        pltpu.async_copy(q2.at[vidx[b]], vrows[b], sem_g.at[b])

    def wait_gather(b):
        pltpu.make_async_copy(
            q2.at[pl.ds(0, 128)], vrows[b], sem_g.at[b]
        ).wait()

    def wait_out(b):
        pltpu.make_async_copy(
            otile[b], o5.at[0, pl.ds(0, 8), pl.ds(0, 1)], sem_o.at[b]
        ).wait()

    fire_idx(0, 0)
    wait_idx(0)
    start_gather(0)
    fire_idx(1, 1)

    def run_block(m, b):
        wait_gather(b)
        bn = 1 - b

        @pl.when(m + 1 < _BPW)
        def _():
            wait_idx(bn)
            start_gather(bn)

        @pl.when(m + 2 < _BPW)
        def _():
            fire_idx(m + 2, b)

        @pl.when(m >= 2)
        def _():
            wait_out(b)

        rows = vrows[b]
        ot = otile[b]

        def tok(di, carry2):
            disp = lanes * 0 + di
            for w in range(4):
                vals = rows[di, pl.ds(w * 16, 16)]
                plsc.store_scatter(ot, [a_idx[w], zeros, bb_idx[w], disp], vals)
            return carry2

        lax.fori_loop(0, 128, tok, 0, unroll=2)

        bid = wid * _BPW + m
        j = bid // 128
        ti = lax.rem(bid, 128)
        pltpu.async_copy(
            ot, o5.at[j, pl.ds(0, 8), pl.ds(ti, 1)], sem_o.at[b]
        )

    def pair(p, carry):
        run_block(2 * p, 0)
        run_block(2 * p + 1, 1)
        return carry

    lax.fori_loop(0, _BPW // 2, pair, 0)
    wait_out(0)
    wait_out(1)


@jax.jit
def kernel(token_ids, weight):
    w3 = weight.reshape(N_TILES, 8, DIM)
    widen = functools.partial(
        pl.kernel,
        out_type=jax.ShapeDtypeStruct((N_TILES, 8, 128), jnp.float32),
        mesh=plsc.VectorSubcoreMesh(core_axis_name="c", subcore_axis_name="s"),
        scratch_types=[
            pltpu.VMEM((_K1_CHUNK, 8, DIM), jnp.float32),
            pltpu.VMEM((_K1_CHUNK, 8, DIM), jnp.float32),
            pltpu.VMEM((_K1_CHUNK, 8, 128), jnp.float32),
            pltpu.VMEM((_K1_CHUNK, 8, 128), jnp.float32),
            pltpu.SemaphoreType.DMA((2,)),
            pltpu.SemaphoreType.DMA((2,)),
        ],
        compiler_params=pltpu.CompilerParams(use_tc_tiling_on_sc=True),
    )(_widen_body)
    q3 = widen(w3)

    idxj = jnp.transpose(token_ids).reshape(B_TOTAL)
    gather = functools.partial(
        pl.kernel,
        out_type=jax.ShapeDtypeStruct((B_J, 8, 128, 8, 128), jnp.float32),
        mesh=plsc.VectorSubcoreMesh(core_axis_name="c", subcore_axis_name="s"),
        scratch_types=[
            pltpu.VMEM((128,), jnp.int32),
            pltpu.VMEM((128,), jnp.int32),
            pltpu.VMEM((128, 128), jnp.float32),
            pltpu.VMEM((128, 128), jnp.float32),
            pltpu.VMEM((8, 1, 8, 128), jnp.float32),
            pltpu.VMEM((8, 1, 8, 128), jnp.float32),
            pltpu.SemaphoreType.DMA((2,)),
            pltpu.SemaphoreType.DMA((2,)),
            pltpu.SemaphoreType.DMA((2,)),
        ],
        compiler_params=pltpu.CompilerParams(
            use_tc_tiling_on_sc=True, needs_layout_passes=False
        ),
    )(_gather_body)
    o5 = gather(idxj, q3.reshape(N_ROWS, 128))
    return o5.transpose(2, 4, 0, 1, 3).reshape(B_I, B_J, DIM)


# speedup vs baseline: 1.4392x; 1.1585x over previous
import functools

import jax
import jax.numpy as jnp
from jax import lax
from jax.experimental import pallas as pl
from jax.experimental.pallas import tpu as pltpu, tpu_sc as plsc

DIM = 64
N_ROWS = 1000000
N_TILES = N_ROWS // 8
B_I, B_J = 16384, 26
B_TOTAL = B_I * B_J

_info = plsc.get_sparse_core_info()
_NC, _NS = _info.num_cores, _info.num_subcores
_NW = _NC * _NS


_K1_BASE = N_TILES // _NW
_K1_CHUNK = 31
_K1_NCH = _K1_BASE // _K1_CHUNK


def _widen_body(w3, q3, vin0, vin1, vout0, vout1, sem_i, sem_o):
    wid = lax.axis_index("s") * _NC + lax.axis_index("c")
    t0 = wid * _K1_BASE
    vin = (vin0, vin1)
    vout = (vout0, vout1)

    def start_in(c, b):
        pltpu.async_copy(
            w3.at[pl.ds(t0 + c * _K1_CHUNK, _K1_CHUNK)], vin[b], sem_i.at[b]
        )

    def wait_in(b):
        pltpu.make_async_copy(
            w3.at[pl.ds(0, _K1_CHUNK)], vin[b], sem_i.at[b]
        ).wait()

    def start_out(c, b):
        pltpu.async_copy(
            vout[b], q3.at[pl.ds(t0 + c * _K1_CHUNK, _K1_CHUNK)], sem_o.at[b]
        )

    def wait_out(b):
        pltpu.make_async_copy(
            vout[b], q3.at[pl.ds(0, _K1_CHUNK)], sem_o.at[b]
        ).wait()

    def widen_chunk(b, nt=_K1_CHUNK):
        vi, vo = vin[b], vout[b]

        @plsc.parallel_loop(0, nt, unroll=2)
        def tile(t):
            for s in range(8):
                for w in range(4):
                    vo[t, s, pl.ds(w * 16, 16)] = vi[t, s, pl.ds(w * 16, 16)]

    start_in(0, 0)

    def pair(p, carry):
        for b in (0, 1):
            c = 2 * p + b
            wait_in(b)

            @pl.when(c + 1 < _K1_NCH)
            def _():
                start_in(c + 1, 1 - b)

            @pl.when(c >= 2)
            def _():
                wait_out(b)

            widen_chunk(b)
            start_out(c, b)
        return carry

    lax.fori_loop(0, _K1_NCH // 2, pair, 0)
    wait_out(0)
    wait_out(1)

    @pl.when(wid == 0)
    def _():
        rem = N_TILES - _NW * _K1_BASE
        pltpu.async_copy(
            w3.at[pl.ds(_NW * _K1_BASE, rem)],
            vin[0].at[pl.ds(0, rem)],
            sem_i.at[0],
        ).wait()
        widen_chunk(0, rem)
        pltpu.async_copy(
            vout[0].at[pl.ds(0, rem)],
            q3.at[pl.ds(_NW * _K1_BASE, rem)],
            sem_o.at[0],
        ).wait()


_BLOCKS = B_J * (B_I // 128)
_BPW = _BLOCKS // _NW


def _gather_body(
    idxj, q2, o5, vidx0, vidx1, vrows0, vrows1, ot0, ot1, sem_i, sem_g, sem_o
):
    wid = lax.axis_index("s") * _NC + lax.axis_index("c")
    vidx = (vidx0, vidx1)
    vrows = (vrows0, vrows1)
    otile = (ot0, ot1)
    lanes = lax.iota(jnp.int32, 16)
    zeros = lanes * 0
    kvecs = [w * 16 + lanes for w in range(4)]
    a_idx = [kv // 8 for kv in kvecs]
    bb_idx = [lax.rem(kv, 8) for kv in kvecs]

    def fire_idx(m, b):
        bid = wid * _BPW + m
        pltpu.async_copy(idxj.at[pl.ds(bid * 128, 128)], vidx[b], sem_i.at[b])

    def wait_idx(b):
        pltpu.make_async_copy(
            idxj.at[pl.ds(0, 128)], vidx[b], sem_i.at[b]
        ).wait()

    def start_gather(b):
        pltpu.async_copy(q2.at[vidx[b]], vrows[b], sem_g.at[b])

    def wait_gather(b):
        pltpu.make_async_copy(
            q2.at[pl.ds(0, 128)], vrows[b], sem_g.at[b]
        ).wait()

    def wait_out(b):
        pltpu.make_async_copy(
            otile[b], o5.at[0, pl.ds(0, 8), pl.ds(0, 1)], sem_o.at[b]
        ).wait()

    fire_idx(0, 0)
    wait_idx(0)
    start_gather(0)
    fire_idx(1, 1)

    def run_block(m, b):
        wait_gather(b)
        bn = 1 - b

        @pl.when(m + 1 < _BPW)
        def _():
            wait_idx(bn)
            start_gather(bn)

        @pl.when(m + 2 < _BPW)
        def _():
            fire_idx(m + 2, b)

        @pl.when(m >= 2)
        def _():
            wait_out(b)

        rows = vrows[b]
        ot = otile[b]

        @plsc.parallel_loop(0, 128, unroll=4)
        def tok(di):
            disp = lanes * 0 + di
            for w in range(4):
                vals = rows[di, pl.ds(w * 16, 16)]
                plsc.store_scatter(ot, [a_idx[w], zeros, bb_idx[w], disp], vals)

        bid = wid * _BPW + m
        j = bid // 128
        ti = lax.rem(bid, 128)
        pltpu.async_copy(
            ot, o5.at[j, pl.ds(0, 8), pl.ds(ti, 1)], sem_o.at[b]
        )

    def pair(p, carry):
        run_block(2 * p, 0)
        run_block(2 * p + 1, 1)
        return carry

    lax.fori_loop(0, _BPW // 2, pair, 0)
    wait_out(0)
    wait_out(1)


@jax.jit
def kernel(token_ids, weight):
    w3 = weight.reshape(N_TILES, 8, DIM)
    widen = functools.partial(
        pl.kernel,
        out_type=jax.ShapeDtypeStruct((N_TILES, 8, 128), jnp.float32),
        mesh=plsc.VectorSubcoreMesh(core_axis_name="c", subcore_axis_name="s"),
        scratch_types=[
            pltpu.VMEM((_K1_CHUNK, 8, DIM), jnp.float32),
            pltpu.VMEM((_K1_CHUNK, 8, DIM), jnp.float32),
            pltpu.VMEM((_K1_CHUNK, 8, 128), jnp.float32),
            pltpu.VMEM((_K1_CHUNK, 8, 128), jnp.float32),
            pltpu.SemaphoreType.DMA((2,)),
            pltpu.SemaphoreType.DMA((2,)),
        ],
        compiler_params=pltpu.CompilerParams(use_tc_tiling_on_sc=True),
    )(_widen_body)
    q3 = widen(w3)

    idxj = jnp.transpose(token_ids).reshape(B_TOTAL)
    gather = functools.partial(
        pl.kernel,
        out_type=jax.ShapeDtypeStruct((B_J, 8, 128, 8, 128), jnp.float32),
        mesh=plsc.VectorSubcoreMesh(core_axis_name="c", subcore_axis_name="s"),
        scratch_types=[
            pltpu.VMEM((128,), jnp.int32),
            pltpu.VMEM((128,), jnp.int32),
            pltpu.VMEM((128, 128), jnp.float32),
            pltpu.VMEM((128, 128), jnp.float32),
            pltpu.VMEM((8, 1, 8, 128), jnp.float32),
            pltpu.VMEM((8, 1, 8, 128), jnp.float32),
            pltpu.SemaphoreType.DMA((2,)),
            pltpu.SemaphoreType.DMA((2,)),
            pltpu.SemaphoreType.DMA((2,)),
        ],
        compiler_params=pltpu.CompilerParams(
            use_tc_tiling_on_sc=True, needs_layout_passes=False
        ),
    )(_gather_body)
    o5 = gather(idxj, q3.reshape(N_ROWS, 128))
    return o5.transpose(2, 4, 0, 1, 3).reshape(B_I, B_J, DIM)


# speedup vs baseline: 1.4891x; 1.0347x over previous
import functools

import jax
import jax.numpy as jnp
from jax import lax
from jax.experimental import pallas as pl
from jax.experimental.pallas import tpu as pltpu, tpu_sc as plsc

DIM = 64
N_ROWS = 1000000
N_TILES = N_ROWS // 8
B_I, B_J = 16384, 26
B_TOTAL = B_I * B_J

_info = plsc.get_sparse_core_info()
_NC, _NS = _info.num_cores, _info.num_subcores
_NW = _NC * _NS


N_PTILES = N_TILES // 2
_K1_BASE = 3904
_K1_CHUNK = 32
_K1_NCH = _K1_BASE // _K1_CHUNK
_K1_TAIL = N_TILES - _NW * _K1_BASE


def _widen_body(w3, q3, vin0, vin1, vout0, vout1, sem_i, sem_o):
    wid = lax.axis_index("s") * _NC + lax.axis_index("c")
    t0 = wid * _K1_BASE
    vin = (vin0, vin1)
    vout = (vout0, vout1)

    def start_in(t_src, b, nt=_K1_CHUNK):
        pltpu.async_copy(
            w3.at[pl.ds(t_src, nt)], vin[b].at[pl.ds(0, nt)], sem_i.at[b]
        )

    def wait_in(b, nt=_K1_CHUNK):
        pltpu.make_async_copy(
            w3.at[pl.ds(0, nt)], vin[b].at[pl.ds(0, nt)], sem_i.at[b]
        ).wait()

    def start_out(t_src, b, nt=_K1_CHUNK):
        pltpu.async_copy(
            vout[b].at[pl.ds(0, nt // 2)],
            q3.at[pl.ds(t_src // 2, nt // 2)],
            sem_o.at[b],
        )

    def wait_out(b, nt=_K1_CHUNK):
        pltpu.make_async_copy(
            vout[b].at[pl.ds(0, nt // 2)],
            q3.at[pl.ds(0, nt // 2)],
            sem_o.at[b],
        ).wait()

    def pack_chunk(b, npair=_K1_CHUNK // 2):
        vi, vo = vin[b], vout[b]

        @plsc.parallel_loop(0, npair, unroll=2)
        def pairfn(p):
            for st in (0, 1):
                for s8 in range(8):
                    ps = 4 * st + s8 // 2
                    h = (s8 % 2) * 64
                    for w in range(4):
                        vo[p, ps, pl.ds(h + 16 * w, 16)] = vi[
                            2 * p + st, s8, pl.ds(16 * w, 16)
                        ]

    start_in(t0, 0)

    def cpair(p, carry):
        for b in (0, 1):
            c = 2 * p + b
            wait_in(b)

            @pl.when(c + 1 < _K1_NCH)
            def _():
                start_in(t0 + (c + 1) * _K1_CHUNK, 1 - b)

            @pl.when(c >= 2)
            def _():
                wait_out(b)

            pack_chunk(b)
            start_out(t0 + c * _K1_CHUNK, b)
        return carry

    lax.fori_loop(0, _K1_NCH // 2, cpair, 0)
    wait_out(0)
    wait_out(1)

    @pl.when(wid < _K1_TAIL // 8)
    def _():
        t_r = _NW * _K1_BASE + wid * 8
        start_in(t_r, 0, 8)
        wait_in(0, 8)
        pack_chunk(0, 4)
        start_out(t_r, 0, 8)
        wait_out(0, 8)


_BLOCKS = B_J * (B_I // 128)
_BPW = _BLOCKS // _NW


def _gather_body(
    idxj,
    q2,
    o5,
    vidx0,
    vidx1,
    vfid0,
    vfid1,
    sid0,
    sid1,
    vrows0,
    vrows1,
    ot0,
    ot1,
    sem_i,
    sem_g,
    sem_o,
):
    wid = lax.axis_index("s") * _NC + lax.axis_index("c")
    vidx = (vidx0, vidx1)
    vfid = (vfid0, vfid1)
    vhalf = (sid0, sid1)
    vrows = (vrows0, vrows1)
    otile = (ot0, ot1)
    lanes = lax.iota(jnp.int32, 16)
    rowvs = [16 * g + lanes for g in range(8)]

    def fire_idx(m, b):
        bid = wid * _BPW + m
        pltpu.async_copy(idxj.at[pl.ds(bid * 128, 128)], vidx[b], sem_i.at[b])

    def wait_idx(b):
        pltpu.make_async_copy(
            idxj.at[pl.ds(0, 128)], vidx[b], sem_i.at[b]
        ).wait()

    def shift_idx(b):
        for v in range(8):
            t = vidx[b][pl.ds(16 * v, 16)]
            vfid[b][pl.ds(16 * v, 16)] = lax.shift_right_logical(t, 1)
            vhalf[b][pl.ds(16 * v, 16)] = (t & 1) * 64

    def start_gather(b):
        pltpu.async_copy(q2.at[vfid[b]], vrows[b], sem_g.at[b])

    def wait_gather(b):
        pltpu.make_async_copy(
            q2.at[pl.ds(0, 128)], vrows[b], sem_g.at[b]
        ).wait()

    def wait_out(b):
        pltpu.make_async_copy(
            otile[b], o5.at[0, pl.ds(0, 8), pl.ds(0, 1)], sem_o.at[b]
        ).wait()

    fire_idx(0, 0)
    wait_idx(0)
    shift_idx(0)
    start_gather(0)
    fire_idx(1, 1)

    def run_block(m, b):
        wait_gather(b)
        bn = 1 - b

        @pl.when(m + 1 < _BPW)
        def _():
            wait_idx(bn)
            shift_idx(bn)
            start_gather(bn)

        @pl.when(m >= 2)
        def _():
            wait_out(b)

        rows = vrows[b]
        ot = otile[b]

        halfs = [vhalf[b][pl.ds(16 * g, 16)] for g in range(8)]

        @plsc.parallel_loop(0, 64, unroll=2)
        def col(k):
            a = k // 8
            bb = lax.rem(k, 8)
            for g in range(8):
                vals = plsc.load_gather(rows, [rowvs[g], halfs[g] + k])
                ot[a, 0, bb, pl.ds(16 * g, 16)] = vals

        bid = wid * _BPW + m
        j = bid // 128
        ti = lax.rem(bid, 128)
        pltpu.async_copy(
            ot, o5.at[j, pl.ds(0, 8), pl.ds(ti, 1)], sem_o.at[b]
        )

        @pl.when(m + 2 < _BPW)
        def _():
            fire_idx(m + 2, b)

    def pair(p, carry):
        run_block(2 * p, 0)
        run_block(2 * p + 1, 1)
        return carry

    lax.fori_loop(0, _BPW // 2, pair, 0)
    wait_out(0)
    wait_out(1)


@jax.jit
def kernel(token_ids, weight):
    w3 = weight.reshape(N_TILES, 8, DIM)
    widen = functools.partial(
        pl.kernel,
        out_type=jax.ShapeDtypeStruct((N_PTILES, 8, 128), jnp.float32),
        mesh=plsc.VectorSubcoreMesh(core_axis_name="c", subcore_axis_name="s"),
        scratch_types=[
            pltpu.VMEM((_K1_CHUNK, 8, DIM), jnp.float32),
            pltpu.VMEM((_K1_CHUNK, 8, DIM), jnp.float32),
            pltpu.VMEM((_K1_CHUNK // 2, 8, 128), jnp.float32),
            pltpu.VMEM((_K1_CHUNK // 2, 8, 128), jnp.float32),
            pltpu.SemaphoreType.DMA((2,)),
            pltpu.SemaphoreType.DMA((2,)),
        ],
        compiler_params=pltpu.CompilerParams(use_tc_tiling_on_sc=True),
    )(_widen_body)
    q3 = widen(w3)

    idxj = jnp.transpose(token_ids).reshape(B_TOTAL)
    gather = functools.partial(
        pl.kernel,
        out_type=jax.ShapeDtypeStruct((B_J, 8, 128, 8, 128), jnp.float32),
        mesh=plsc.VectorSubcoreMesh(core_axis_name="c", subcore_axis_name="s"),
        scratch_types=[
            pltpu.VMEM((128,), jnp.int32),
            pltpu.VMEM((128,), jnp.int32),
            pltpu.VMEM((128,), jnp.int32),
            pltpu.VMEM((128,), jnp.int32),
            pltpu.VMEM((128,), jnp.int32),
            pltpu.VMEM((128,), jnp.int32),
            pltpu.VMEM((128, 128), jnp.float32),
            pltpu.VMEM((128, 128), jnp.float32),
            pltpu.VMEM((8, 1, 8, 128), jnp.float32),
            pltpu.VMEM((8, 1, 8, 128), jnp.float32),
            pltpu.SemaphoreType.DMA((2,)),
            pltpu.SemaphoreType.DMA((2,)),
            pltpu.SemaphoreType.DMA((2,)),
        ],
        compiler_params=pltpu.CompilerParams(
            use_tc_tiling_on_sc=True, needs_layout_passes=False
        ),
    )(_gather_body)
    o5 = gather(idxj, q3.reshape(N_ROWS // 2, 128))
    return o5.transpose(2, 4, 0, 1, 3).reshape(B_I, B_J, DIM)


# speedup vs baseline: 1.4924x; 1.0022x over previous
import functools

import jax
import jax.numpy as jnp
from jax import lax
from jax.experimental import pallas as pl
from jax.experimental.pallas import tpu as pltpu, tpu_sc as plsc

DIM = 64
N_ROWS = 1000000
N_TILES = N_ROWS // 8
B_I, B_J = 16384, 26
B_TOTAL = B_I * B_J

_info = plsc.get_sparse_core_info()
_NC, _NS = _info.num_cores, _info.num_subcores
_NW = _NC * _NS


N_PTILES = N_TILES // 2
_K1_BASE = 3904
_K1_CHUNK = 32
_K1_NCH = _K1_BASE // _K1_CHUNK
_K1_TAIL = N_TILES - _NW * _K1_BASE


def _widen_body(w3, q3, vin0, vin1, vout0, vout1, sem_i, sem_o):
    wid = lax.axis_index("s") * _NC + lax.axis_index("c")
    t0 = wid * _K1_BASE
    vin = (vin0, vin1)
    vout = (vout0, vout1)

    def start_in(t_src, b, nt=_K1_CHUNK):
        pltpu.async_copy(
            w3.at[pl.ds(t_src, nt)], vin[b].at[pl.ds(0, nt)], sem_i.at[b]
        )

    def wait_in(b, nt=_K1_CHUNK):
        pltpu.make_async_copy(
            w3.at[pl.ds(0, nt)], vin[b].at[pl.ds(0, nt)], sem_i.at[b]
        ).wait()

    def start_out(t_src, b, nt=_K1_CHUNK):
        pltpu.async_copy(
            vout[b].at[pl.ds(0, nt // 2)],
            q3.at[pl.ds(t_src // 2, nt // 2)],
            sem_o.at[b],
        )

    def wait_out(b, nt=_K1_CHUNK):
        pltpu.make_async_copy(
            vout[b].at[pl.ds(0, nt // 2)],
            q3.at[pl.ds(0, nt // 2)],
            sem_o.at[b],
        ).wait()

    def pack_chunk(b, npair=_K1_CHUNK // 2):
        vi, vo = vin[b], vout[b]

        @plsc.parallel_loop(0, npair, unroll=2)
        def pairfn(p):
            for st in (0, 1):
                for s8 in range(8):
                    ps = 4 * st + s8 // 2
                    h = (s8 % 2) * 64
                    for w in range(4):
                        vo[p, ps, pl.ds(h + 16 * w, 16)] = vi[
                            2 * p + st, s8, pl.ds(16 * w, 16)
                        ]

    start_in(t0, 0)

    def cpair(p, carry):
        for b in (0, 1):
            c = 2 * p + b
            wait_in(b)

            @pl.when(c + 1 < _K1_NCH)
            def _():
                start_in(t0 + (c + 1) * _K1_CHUNK, 1 - b)

            @pl.when(c >= 2)
            def _():
                wait_out(b)

            pack_chunk(b)
            start_out(t0 + c * _K1_CHUNK, b)
        return carry

    lax.fori_loop(0, _K1_NCH // 2, cpair, 0)
    wait_out(0)
    wait_out(1)

    @pl.when(wid < _K1_TAIL // 8)
    def _():
        t_r = _NW * _K1_BASE + wid * 8
        start_in(t_r, 0, 8)
        wait_in(0, 8)
        pack_chunk(0, 4)
        start_out(t_r, 0, 8)
        wait_out(0, 8)


_BLOCKS = B_J * (B_I // 128)
_BPW = _BLOCKS // _NW


def _gather_body(
    idxj,
    q2,
    o5,
    vidx0,
    vidx1,
    vfid0,
    vfid1,
    sid0,
    sid1,
    vrows0,
    vrows1,
    ot0,
    ot1,
    sem_i,
    sem_g,
    sem_o,
):
    wid = lax.axis_index("s") * _NC + lax.axis_index("c")
    vidx = (vidx0, vidx1)
    vfid = (vfid0, vfid1)
    vhalf = (sid0, sid1)
    vrows = (vrows0, vrows1)
    otile = (ot0, ot1)
    lanes = lax.iota(jnp.int32, 16)
    rowvs = [16 * g + lanes for g in range(8)]

    def fire_idx(m, b):
        bid = wid * _BPW + m
        pltpu.async_copy(idxj.at[pl.ds(bid * 128, 128)], vidx[b], sem_i.at[b])

    def wait_idx(b):
        pltpu.make_async_copy(
            idxj.at[pl.ds(0, 128)], vidx[b], sem_i.at[b]
        ).wait()

    def shift_idx(b):
        for v in range(8):
            t = vidx[b][pl.ds(16 * v, 16)]
            vfid[b][pl.ds(16 * v, 16)] = lax.shift_right_logical(t, 1)
            vhalf[b][pl.ds(16 * v, 16)] = (t & 1) * 64

    def start_gather(b):
        pltpu.async_copy(q2.at[vfid[b]], vrows[b], sem_g.at[b])

    def wait_gather(b):
        pltpu.make_async_copy(
            q2.at[pl.ds(0, 128)], vrows[b], sem_g.at[b]
        ).wait()

    def wait_out(b):
        pltpu.make_async_copy(
            otile[b], o5.at[0, pl.ds(0, 8), pl.ds(0, 1)], sem_o.at[b]
        ).wait()

    fire_idx(0, 0)
    wait_idx(0)
    shift_idx(0)
    start_gather(0)
    fire_idx(1, 1)

    def run_block(m, b):
        wait_gather(b)
        bn = 1 - b

        @pl.when(m + 1 < _BPW)
        def _():
            wait_idx(bn)
            shift_idx(bn)
            start_gather(bn)

        @pl.when(m >= 2)
        def _():
            wait_out(b)

        rows = vrows[b]
        ot = otile[b]

        halfs = [vhalf[b][pl.ds(16 * g, 16)] for g in range(8)]

        @plsc.parallel_loop(0, 64, unroll=4)
        def col(k):
            a = k // 8
            bb = lax.rem(k, 8)
            for g in range(8):
                vals = plsc.load_gather(rows, [rowvs[g], halfs[g] + k])
                ot[a, 0, bb, pl.ds(16 * g, 16)] = vals

        bid = wid * _BPW + m
        j = bid // 128
        ti = lax.rem(bid, 128)
        pltpu.async_copy(
            ot, o5.at[j, pl.ds(0, 8), pl.ds(ti, 1)], sem_o.at[b]
        )

        @pl.when(m + 2 < _BPW)
        def _():
            fire_idx(m + 2, b)

    def pair(p, carry):
        run_block(2 * p, 0)
        run_block(2 * p + 1, 1)
        return carry

    lax.fori_loop(0, _BPW // 2, pair, 0)
    wait_out(0)
    wait_out(1)


@jax.jit
def kernel(token_ids, weight):
    w3 = weight.reshape(N_TILES, 8, DIM)
    widen = functools.partial(
        pl.kernel,
        out_type=jax.ShapeDtypeStruct((N_PTILES, 8, 128), jnp.float32),
        mesh=plsc.VectorSubcoreMesh(core_axis_name="c", subcore_axis_name="s"),
        scratch_types=[
            pltpu.VMEM((_K1_CHUNK, 8, DIM), jnp.float32),
            pltpu.VMEM((_K1_CHUNK, 8, DIM), jnp.float32),
            pltpu.VMEM((_K1_CHUNK // 2, 8, 128), jnp.float32),
            pltpu.VMEM((_K1_CHUNK // 2, 8, 128), jnp.float32),
            pltpu.SemaphoreType.DMA((2,)),
            pltpu.SemaphoreType.DMA((2,)),
        ],
        compiler_params=pltpu.CompilerParams(use_tc_tiling_on_sc=True),
    )(_widen_body)
    q3 = widen(w3)

    idxj = jnp.transpose(token_ids).reshape(B_TOTAL)
    gather = functools.partial(
        pl.kernel,
        out_type=jax.ShapeDtypeStruct((B_J, 8, 128, 8, 128), jnp.float32),
        mesh=plsc.VectorSubcoreMesh(core_axis_name="c", subcore_axis_name="s"),
        scratch_types=[
            pltpu.VMEM((128,), jnp.int32),
            pltpu.VMEM((128,), jnp.int32),
            pltpu.VMEM((128,), jnp.int32),
            pltpu.VMEM((128,), jnp.int32),
            pltpu.VMEM((128,), jnp.int32),
            pltpu.VMEM((128,), jnp.int32),
            pltpu.VMEM((128, 128), jnp.float32),
            pltpu.VMEM((128, 128), jnp.float32),
            pltpu.VMEM((8, 1, 8, 128), jnp.float32),
            pltpu.VMEM((8, 1, 8, 128), jnp.float32),
            pltpu.SemaphoreType.DMA((2,)),
            pltpu.SemaphoreType.DMA((2,)),
            pltpu.SemaphoreType.DMA((2,)),
        ],
        compiler_params=pltpu.CompilerParams(
            use_tc_tiling_on_sc=True, needs_layout_passes=False
        ),
    )(_gather_body)
    o5 = gather(idxj, q3.reshape(N_ROWS // 2, 128))
    return o5.transpose(2, 4, 0, 1, 3).reshape(B_I, B_J, DIM)
